# baseline (speedup 1.0000x reference)
import functools

import jax
import jax.numpy as jnp
from jax import lax
from jax.experimental import pallas as pl
from jax.experimental.pallas import tpu as pltpu
from jax.experimental.pallas import tpu_sc as plsc

VOCAB = 1000000
EMBED = 32
NUM_CORES = 2
NUM_SUBCORES = 16
NUM_WORKERS = NUM_CORES * NUM_SUBCORES
L = 16
BB = 128
TQ = 4


@functools.partial(jax.jit, static_argnums=(2, 3))
def _gather_embed(tok, table, n_pos, n_batch):
    mesh = plsc.VectorSubcoreMesh(core_axis_name="c", subcore_axis_name="s")
    n_units = n_pos // TQ
    assert n_units % 2 == 0 and n_batch // BB == NUM_WORKERS
    GB = TQ * BB

    @functools.partial(
        pl.kernel,
        mesh=mesh,
        out_type=jax.ShapeDtypeStruct((n_pos, EMBED // 8, n_batch * 8),
                                      jnp.float32),
        scratch_types=[
            pltpu.VMEM((BB, n_pos), jnp.float32),
            pltpu.VMEM((n_pos * BB,), jnp.int32),
        ] + [pltpu.VMEM((GB, EMBED), jnp.float32)] * 2
          + [pltpu.VMEM((TQ, EMBED // 8, 8 * BB), jnp.float32)] * 2
          + [pltpu.SemaphoreType.DMA] * 4,
        compiler_params=pltpu.CompilerParams(
            use_tc_tiling_on_sc=False, needs_layout_passes=False),
    )
    def k(tok_hbm, table_hbm, out_hbm, slab, idxs, r0, r1, t0, t1,
          sg0, sg1, so0, so1):
        rows = (r0, r1)
        tbuf = (t0, t1)
        sg = (sg0, sg1)
        so = (so0, so1)
        wid = lax.axis_index("s") * NUM_CORES + lax.axis_index("c")
        b0 = wid * BB

        iot = lax.iota(jnp.int32, L)
        bvecs = [iot + L * j for j in range(BB // L)]

        pltpu.sync_copy(tok_hbm.at[pl.ds(b0, BB), :], slab)

        def slab_body(i, c):
            vs = []
            for jj in range(2 * BB // L):
                tvec = jnp.full((L,), 0, jnp.int32) + (2 * i + jj // 8)
                vs.append(plsc.load_gather(slab, [bvecs[jj % 8], tvec]))
            for jj in range(2 * BB // L):
                idxs[pl.ds(i * 2 * BB + jj * L, L)] = \
                    plsc.bitcast(vs[jj], jnp.int32)
            return c
        lax.fori_loop(0, n_pos // 2, slab_body, 0)

        def gather_start(u, rb):
            pltpu.async_copy(table_hbm.at[idxs.at[pl.ds(u * GB, GB)]],
                             rows[rb], sg[rb])

        def gather_wait(u, rb):
            pltpu.make_async_copy(table_hbm.at[idxs.at[pl.ds(u * GB, GB)]],
                                  rows[rb], sg[rb]).wait()

        def wb_start(u, rb):
            for tl in range(TQ):
                pltpu.async_copy(
                    tbuf[rb].at[tl],
                    out_hbm.at[u * TQ + tl, :, pl.ds(b0 * 8, 8 * BB)],
                    so[rb])

        def wb_wait(u, rb):
            for tl in range(TQ):
                pltpu.make_async_copy(
                    tbuf[rb].at[tl],
                    out_hbm.at[u * TQ + tl, :, pl.ds(b0 * 8, 8 * BB)],
                    so[rb]).wait()

        def transpose(rb):
            tb = tbuf[rb]
            def e_body(e, c):
                evec = jnp.full((L,), 0, jnp.int32) + e
                vs = []
                for j in range(GB // L):
                    vs.append(plsc.load_gather(rows[rb], [iot + L * j, evec]))
                for j in range(GB // L):
                    tb[j // 8, e // 8, pl.ds((e % 8) * BB + (j % 8) * L, L)] \
                        = vs[j]
                return c
            lax.fori_loop(0, EMBED, e_body, 0)

        gather_start(0, 0)

        def body(kk, carry):
            for rb in range(2):
                i = 2 * kk + rb
                if rb == 0:
                    gather_start(i + 1, 1)
                else:
                    @pl.when(kk < n_units // 2 - 1)
                    def _():
                        gather_start(i + 1, 0)
                gather_wait(i, rb)
                @pl.when(kk > 0)
                def _():
                    wb_wait(i - 2, rb)
                transpose(rb)
                wb_start(i, rb)
            return carry

        lax.fori_loop(0, n_units // 2, body, 0)
        wb_wait(n_units - 2, 0)
        wb_wait(n_units - 1, 1)

    return k(tok, table)


def kernel(tokens, embedding_weight):
    n_batch, n_pos = tokens.shape
    tok_f = jax.lax.bitcast_convert_type(tokens.astype(jnp.int32),
                                         jnp.float32)
    out3 = _gather_embed(tok_f, embedding_weight, n_pos, n_batch)
    out = (out3.reshape(n_pos, EMBED // 8, n_batch // 128, 8, 128)
           .transpose(2, 4, 0, 1, 3)
           .reshape(n_batch, n_pos, EMBED))
    return out


# speedup vs baseline: 1.0011x; 1.0011x over previous
import functools

import jax
import jax.numpy as jnp
from jax import lax
from jax.experimental import pallas as pl
from jax.experimental.pallas import tpu as pltpu
from jax.experimental.pallas import tpu_sc as plsc

VOCAB = 1000000
EMBED = 32
NUM_CORES = 2
NUM_SUBCORES = 16
NUM_WORKERS = NUM_CORES * NUM_SUBCORES
L = 16
BB = 128
TQ = 4


@functools.partial(jax.jit, static_argnums=(2, 3))
def _gather_embed(tok, table, n_pos, n_batch):
    mesh = plsc.VectorSubcoreMesh(core_axis_name="c", subcore_axis_name="s")
    n_units = n_pos // TQ
    assert n_units % 2 == 0 and n_batch // BB == NUM_WORKERS
    GB = TQ * BB

    @functools.partial(
        pl.kernel,
        mesh=mesh,
        out_type=jax.ShapeDtypeStruct((n_pos, EMBED // 8, n_batch * 8),
                                      jnp.float32),
        scratch_types=[
            pltpu.VMEM((BB, n_pos), jnp.int32),
            pltpu.VMEM((n_pos * BB,), jnp.int32),
        ] + [pltpu.VMEM((GB, EMBED), jnp.float32)] * 2
          + [pltpu.VMEM((TQ, EMBED // 8, 8 * BB), jnp.float32)] * 2
          + [pltpu.SemaphoreType.DMA] * 4,
        compiler_params=pltpu.CompilerParams(
            use_tc_tiling_on_sc=False, needs_layout_passes=False),
    )
    def k(tok_hbm, table_hbm, out_hbm, slab, idxs, r0, r1, t0, t1,
          sg0, sg1, so0, so1):
        rows = (r0, r1)
        tbuf = (t0, t1)
        sg = (sg0, sg1)
        so = (so0, so1)
        wid = lax.axis_index("s") * NUM_CORES + lax.axis_index("c")
        b0 = wid * BB

        iot = lax.iota(jnp.int32, L)
        bvecs = [iot + L * j for j in range(BB // L)]

        pltpu.sync_copy(tok_hbm.at[pl.ds(b0, BB), :], slab)

        def slab_body(i, c):
            vs = []
            for jj in range(2 * BB // L):
                tvec = jnp.full((L,), 0, jnp.int32) + (2 * i + jj // 8)
                vs.append(plsc.load_gather(slab, [bvecs[jj % 8], tvec]))
            for jj in range(2 * BB // L):
                idxs[pl.ds(i * 2 * BB + jj * L, L)] = vs[jj]
            return c
        lax.fori_loop(0, n_pos // 2, slab_body, 0)

        def gather_start(u, rb):
            pltpu.async_copy(table_hbm.at[idxs.at[pl.ds(u * GB, GB)]],
                             rows[rb], sg[rb])

        def gather_wait(u, rb):
            pltpu.make_async_copy(table_hbm.at[idxs.at[pl.ds(u * GB, GB)]],
                                  rows[rb], sg[rb]).wait()

        def wb_start(u, rb):
            for tl in range(TQ):
                pltpu.async_copy(
                    tbuf[rb].at[tl],
                    out_hbm.at[u * TQ + tl, :, pl.ds(b0 * 8, 8 * BB)],
                    so[rb])

        def wb_wait(u, rb):
            for tl in range(TQ):
                pltpu.make_async_copy(
                    tbuf[rb].at[tl],
                    out_hbm.at[u * TQ + tl, :, pl.ds(b0 * 8, 8 * BB)],
                    so[rb]).wait()

        def transpose(rb):
            tb = tbuf[rb]
            def e_body(e, c):
                evec = jnp.full((L,), 0, jnp.int32) + e
                vs = []
                for j in range(GB // L):
                    vs.append(plsc.load_gather(rows[rb], [iot + L * j, evec]))
                for j in range(GB // L):
                    tb[j // 8, e // 8, pl.ds((e % 8) * BB + (j % 8) * L, L)] \
                        = vs[j]
                return c
            lax.fori_loop(0, EMBED, e_body, 0)

        gather_start(0, 0)

        def body(kk, carry):
            for rb in range(2):
                i = 2 * kk + rb
                if rb == 0:
                    gather_start(i + 1, 1)
                else:
                    @pl.when(kk < n_units // 2 - 1)
                    def _():
                        gather_start(i + 1, 0)
                gather_wait(i, rb)
                @pl.when(kk > 0)
                def _():
                    wb_wait(i - 2, rb)
                transpose(rb)
                wb_start(i, rb)
            return carry

        lax.fori_loop(0, n_units // 2, body, 0)
        wb_wait(n_units - 2, 0)
        wb_wait(n_units - 1, 1)

    return k(tok, table)


def kernel(tokens, embedding_weight):
    n_batch, n_pos = tokens.shape
    out3 = _gather_embed(tokens, embedding_weight, n_pos, n_batch)
    out = (out3.reshape(n_pos, EMBED // 8, n_batch // 128, 8, 128)
           .transpose(2, 4, 0, 1, 3)
           .reshape(n_batch, n_pos, EMBED))
    return out


# speedup vs baseline: 1.7056x; 1.7037x over previous
import functools

import jax
import jax.numpy as jnp
from jax import lax
from jax.experimental import pallas as pl
from jax.experimental.pallas import tpu as pltpu
from jax.experimental.pallas import tpu_sc as plsc

VOCAB = 1000000
EMBED = 32
NUM_CORES = 2
NUM_SUBCORES = 16
NUM_WORKERS = NUM_CORES * NUM_SUBCORES
L = 16
BB = 128
TQ = 4


@functools.partial(jax.jit, static_argnums=(2, 3))
def _gather_embed(tok, table, n_pos, n_batch):
    mesh = plsc.VectorSubcoreMesh(core_axis_name="c", subcore_axis_name="s")
    n_units = n_pos // TQ
    assert n_units % 2 == 0 and n_batch // BB == NUM_WORKERS
    GB = TQ * BB

    @functools.partial(
        pl.kernel,
        mesh=mesh,
        out_type=jax.ShapeDtypeStruct((n_pos, EMBED // 8, n_batch * 8),
                                      jnp.float32),
        scratch_types=[
            pltpu.VMEM((BB, n_pos), jnp.int32),
            pltpu.VMEM((n_pos * BB,), jnp.int32),
        ] + [pltpu.VMEM((GB, EMBED), jnp.float32)] * 2
          + [pltpu.VMEM((TQ * EMBED * BB,), jnp.float32)] * 2
          + [pltpu.SemaphoreType.DMA] * 4,
        compiler_params=pltpu.CompilerParams(
            use_tc_tiling_on_sc=False, needs_layout_passes=False),
    )
    def k(tok_hbm, table_hbm, out_hbm, slab, idxs, r0, r1, t0, t1,
          sg0, sg1, so0, so1):
        rows = (r0, r1)
        tbuf = (t0, t1)
        sg = (sg0, sg1)
        so = (so0, so1)
        wid = lax.axis_index("s") * NUM_CORES + lax.axis_index("c")
        b0 = wid * BB

        iot = lax.iota(jnp.int32, L)
        bvecs = [iot + L * j for j in range(BB // L)]

        pltpu.sync_copy(tok_hbm.at[pl.ds(b0, BB), :], slab)

        def slab_body(i, c):
            vs = []
            for jj in range(2 * BB // L):
                tvec = jnp.full((L,), 0, jnp.int32) + (2 * i + jj // 8)
                vs.append(plsc.load_gather(slab, [bvecs[jj % 8], tvec]))
            for jj in range(2 * BB // L):
                idxs[pl.ds(i * 2 * BB + jj * L, L)] = vs[jj]
            return c
        lax.fori_loop(0, n_pos // 2, slab_body, 0)

        def gather_start(u, rb):
            pltpu.async_copy(table_hbm.at[idxs.at[pl.ds(u * GB, GB)]],
                             rows[rb], sg[rb])

        def gather_wait(u, rb):
            pltpu.make_async_copy(table_hbm.at[idxs.at[pl.ds(u * GB, GB)]],
                                  rows[rb], sg[rb]).wait()

        def wb_start(u, rb):
            for tl in range(TQ):
                for e8 in range(EMBED // 8):
                    pltpu.async_copy(
                        tbuf[rb].at[pl.ds((tl * 4 + e8) * 8 * BB, 8 * BB)],
                        out_hbm.at[u * TQ + tl, e8, pl.ds(b0 * 8, 8 * BB)],
                        so[rb])

        def wb_wait(u, rb):
            for tl in range(TQ):
                for e8 in range(EMBED // 8):
                    pltpu.make_async_copy(
                        tbuf[rb].at[pl.ds((tl * 4 + e8) * 8 * BB, 8 * BB)],
                        out_hbm.at[u * TQ + tl, e8, pl.ds(b0 * 8, 8 * BB)],
                        so[rb]).wait()

        perms = [(iot + d) & 15 for d in range(L)]
        fdst = [(perms[d] // 8) * 1024 + (perms[d] % 8) * BB + iot
                for d in range(L)]

        def transpose(rb):
            tb = tbuf[rb]
            def jb_body(jb, c):
                bv = iot + jb * L
                for eh in range(2):
                    sb_dst = ((jb // 8) * (4 * 8 * BB) + (eh * 2) * (8 * BB)
                              + (jb % 8) * L)
                    vs = []
                    for d in range(L):
                        vs.append(plsc.load_gather(
                            rows[rb], [bv, perms[d] + eh * L]))
                    for d in range(L):
                        plsc.store_scatter(tb, [fdst[d] + sb_dst], vs[d])
                return c
            lax.fori_loop(0, GB // L, jb_body, 0)

        gather_start(0, 0)

        def body(kk, carry):
            for rb in range(2):
                i = 2 * kk + rb
                if rb == 0:
                    gather_start(i + 1, 1)
                else:
                    @pl.when(kk < n_units // 2 - 1)
                    def _():
                        gather_start(i + 1, 0)
                gather_wait(i, rb)
                @pl.when(kk > 0)
                def _():
                    wb_wait(i - 2, rb)
                transpose(rb)
                wb_start(i, rb)
            return carry

        lax.fori_loop(0, n_units // 2, body, 0)
        wb_wait(n_units - 2, 0)
        wb_wait(n_units - 1, 1)

    return k(tok, table)


def kernel(tokens, embedding_weight):
    n_batch, n_pos = tokens.shape
    out3 = _gather_embed(tokens, embedding_weight, n_pos, n_batch)
    out = (out3.reshape(n_pos, EMBED // 8, n_batch // 128, 8, 128)
           .transpose(2, 4, 0, 1, 3)
           .reshape(n_batch, n_pos, EMBED))
    return out


# speedup vs baseline: 1.7166x; 1.0065x over previous
import functools

import jax
import jax.numpy as jnp
from jax import lax
from jax.experimental import pallas as pl
from jax.experimental.pallas import tpu as pltpu
from jax.experimental.pallas import tpu_sc as plsc

VOCAB = 1000000
EMBED = 32
NUM_CORES = 2
NUM_SUBCORES = 16
NUM_WORKERS = NUM_CORES * NUM_SUBCORES
L = 16
BB = 128
TQ = 4


@functools.partial(jax.jit, static_argnums=(1, 2))
def _transpose_tokens(tok, n_pos, n_batch):
    mesh = plsc.VectorSubcoreMesh(core_axis_name="c", subcore_axis_name="s")

    @functools.partial(
        pl.kernel,
        mesh=mesh,
        out_type=jax.ShapeDtypeStruct((NUM_WORKERS, n_pos // TQ, TQ * BB),
                                      jnp.int32),
        scratch_types=[
            pltpu.VMEM((BB, n_pos), jnp.int32),
            pltpu.VMEM((n_pos // TQ, TQ * BB), jnp.int32),
        ],
        compiler_params=pltpu.CompilerParams(
            use_tc_tiling_on_sc=True, needs_layout_passes=False),
    )
    def k(tok_hbm, out_hbm, slab, tbuf):
        wid = lax.axis_index("s") * NUM_CORES + lax.axis_index("c")
        b0 = wid * BB
        iot = lax.iota(jnp.int32, L)
        bvecs = [iot + L * j for j in range(BB // L)]
        pltpu.sync_copy(tok_hbm.at[pl.ds(b0, BB), :], slab)

        def slab_body(i, c):
            vs = []
            for jj in range(2 * BB // L):
                tvec = jnp.full((L,), 0, jnp.int32) + (2 * i + jj // 8)
                vs.append(plsc.load_gather(slab, [bvecs[jj % 8], tvec]))
            for jj in range(2 * BB // L):
                t = 2 * i + jj // 8
                tbuf[t // TQ, pl.ds((t % TQ) * BB + (jj % 8) * L, L)] = vs[jj]
            return c
        lax.fori_loop(0, n_pos // 2, slab_body, 0)
        pltpu.sync_copy(tbuf, out_hbm.at[wid])

    return k(tok)


@functools.partial(jax.jit, static_argnums=(2, 3))
def _gather_embed(idxt, table, n_pos, n_batch):
    mesh = plsc.VectorSubcoreMesh(core_axis_name="c", subcore_axis_name="s")
    n_units = n_pos // TQ
    assert n_units % 2 == 0 and n_batch // BB == NUM_WORKERS
    GB = TQ * BB

    @functools.partial(
        pl.kernel,
        mesh=mesh,
        out_type=jax.ShapeDtypeStruct((n_pos, EMBED // 8, n_batch * 8),
                                      jnp.float32),
        scratch_types=[
            pltpu.VMEM((n_units, GB), jnp.int32),
        ] + [pltpu.VMEM((GB, EMBED), jnp.float32)] * 2
          + [pltpu.VMEM((TQ * EMBED * BB,), jnp.float32)] * 2
          + [pltpu.SemaphoreType.DMA] * 4,
        compiler_params=pltpu.CompilerParams(
            use_tc_tiling_on_sc=False, needs_layout_passes=False),
    )
    def k(idxt_hbm, table_hbm, out_hbm, idxs, r0, r1, t0, t1,
          sg0, sg1, so0, so1):
        rows = (r0, r1)
        tbuf = (t0, t1)
        sg = (sg0, sg1)
        so = (so0, so1)
        wid = lax.axis_index("s") * NUM_CORES + lax.axis_index("c")
        b0 = wid * BB

        iot = lax.iota(jnp.int32, L)
        pltpu.sync_copy(idxt_hbm.at[wid], idxs)

        def gather_start(u, rb):
            pltpu.async_copy(table_hbm.at[idxs.at[u]], rows[rb], sg[rb])

        def gather_wait(u, rb):
            pltpu.make_async_copy(table_hbm.at[idxs.at[u]],
                                  rows[rb], sg[rb]).wait()

        def wb_start(u, rb):
            for tl in range(TQ):
                for e8 in range(EMBED // 8):
                    pltpu.async_copy(
                        tbuf[rb].at[pl.ds((tl * 4 + e8) * 8 * BB, 8 * BB)],
                        out_hbm.at[u * TQ + tl, e8, pl.ds(b0 * 8, 8 * BB)],
                        so[rb])

        def wb_wait(u, rb):
            for tl in range(TQ):
                for e8 in range(EMBED // 8):
                    pltpu.make_async_copy(
                        tbuf[rb].at[pl.ds((tl * 4 + e8) * 8 * BB, 8 * BB)],
                        out_hbm.at[u * TQ + tl, e8, pl.ds(b0 * 8, 8 * BB)],
                        so[rb]).wait()

        perms = [(iot + d) & 15 for d in range(L)]
        fdst = [(perms[d] // 8) * 1024 + (perms[d] % 8) * BB + iot
                for d in range(L)]

        def transpose(rb):
            tb = tbuf[rb]
            def jb_body(jb, c):
                bv = iot + jb * L
                for eh in range(2):
                    sb_dst = ((jb // 8) * (4 * 8 * BB) + (eh * 2) * (8 * BB)
                              + (jb % 8) * L)
                    vs = []
                    for d in range(L):
                        vs.append(plsc.load_gather(
                            rows[rb], [bv, perms[d] + eh * L]))
                    for d in range(L):
                        plsc.store_scatter(tb, [fdst[d] + sb_dst], vs[d])
                return c
            lax.fori_loop(0, GB // L, jb_body, 0)

        gather_start(0, 0)

        def body(kk, carry):
            for rb in range(2):
                i = 2 * kk + rb
                if rb == 0:
                    gather_start(i + 1, 1)
                else:
                    @pl.when(kk < n_units // 2 - 1)
                    def _():
                        gather_start(i + 1, 0)
                gather_wait(i, rb)
                @pl.when(kk > 0)
                def _():
                    wb_wait(i - 2, rb)
                transpose(rb)
                wb_start(i, rb)
            return carry

        lax.fori_loop(0, n_units // 2, body, 0)
        wb_wait(n_units - 2, 0)
        wb_wait(n_units - 1, 1)

    return k(idxt, table)


def kernel(tokens, embedding_weight):
    n_batch, n_pos = tokens.shape
    idxt = _transpose_tokens(tokens, n_pos, n_batch)
    out3 = _gather_embed(idxt, embedding_weight, n_pos, n_batch)
    out = (out3.reshape(n_pos, EMBED // 8, n_batch // 128, 8, 128)
           .transpose(2, 4, 0, 1, 3)
           .reshape(n_batch, n_pos, EMBED))
    return out
